# baseline (speedup 1.0000x reference)
import jax
import jax.numpy as jnp
import numpy as np
from jax import lax
from jax.experimental import pallas as pl
from jax.experimental.pallas import tpu as pltpu
from jax.experimental.pallas import tpu_sc as plsc

N = 10000
E = 160000
NP = 10240
D = 256
DH = 128
ER = 2000
EC = 80
SR = 25
NSC = 5
RPT = ER // 16
NRT = NP // 16
F32 = jnp.float32
BF16 = jnp.bfloat16

_SC_PARAMS = pltpu.CompilerParams(use_tc_tiling_on_sc=False,
                                  needs_layout_passes=False)

_PERM = np.concatenate(
    [np.concatenate([np.arange(0, 32, 2), np.arange(1, 32, 2)]) + 32 * k
     for k in range(DH // 32)])


def _newton_rsqrt(d):
    i = plsc.bitcast(d, jnp.int32)
    i = jnp.int32(0x5F3759DF) - lax.shift_right_logical(i, 1)
    y = plsc.bitcast(i, F32)
    for _ in range(4):
        y = y * (1.5 - 0.5 * d * y * y)
    return y


def _sca_body(col_hbm, ew_hbm, dinv_out,
              col_l, ew_l, tmp_l, dd_sh, sem):
    c = lax.axis_index("c")
    s = lax.axis_index("s")

    def ones_step(i, _):
        tmp_l[pl.ds(i * 16, 16)] = jnp.full((16,), 1.0, F32)
        return 0
    lax.fori_loop(0, NRT // 16, ones_step, 0)
    pltpu.sync_copy(tmp_l, dd_sh.at[pl.ds(s * NRT, NRT)])
    plsc.subcore_barrier()

    def deg_chunk(g, _):
        gbase = s * RPT + g * SR
        pltpu.sync_copy(col_hbm.at[pl.ds(gbase, SR)], col_l)
        pltpu.sync_copy(ew_hbm.at[pl.ds(gbase, SR)], ew_l)

        def deg_fire(j, _):
            pltpu.async_copy(ew_l.at[j], dd_sh.at[col_l.at[j]],
                             sem, add=True)
            return 0
        lax.fori_loop(0, SR, deg_fire, 0)

        def deg_drain(j, _):
            pltpu.make_async_copy(ew_l.at[j], dd_sh.at[pl.ds(0, EC)],
                                  sem).wait()
            return 0
        lax.fori_loop(0, SR, deg_drain, 0)
        return 0
    lax.fori_loop(0, NSC, deg_chunk, 0)
    plsc.subcore_barrier()

    pltpu.sync_copy(dd_sh.at[pl.ds(s * NRT, NRT)], tmp_l)

    def dinv_step(i, _):
        d = tmp_l[pl.ds(i * 16, 16)]
        tmp_l[pl.ds(i * 16, 16)] = _newton_rsqrt(d)
        return 0
    lax.fori_loop(0, NRT // 16, dinv_step, 0)

    @pl.when(c == 0)
    def _():
        pltpu.sync_copy(tmp_l, dinv_out.at[s])


def _sc_deg(col2, ew2):
    mesh = plsc.VectorSubcoreMesh(core_axis_name="c", subcore_axis_name="s")
    f = pl.kernel(
        _sca_body,
        out_type=jax.ShapeDtypeStruct((16, NRT), F32),
        mesh=mesh,
        compiler_params=_SC_PARAMS,
        scratch_types=[
            pltpu.VMEM((SR, EC), jnp.int32),
            pltpu.VMEM((SR, EC), F32),
            pltpu.VMEM((NRT,), F32),
            pltpu.VMEM_SHARED((NP,), F32),
            pltpu.SemaphoreType.DMA,
        ],
    )
    return f(col2, ew2)


def _mm_body(x_ref, dinv_ref, w_ref, o_ref):
    r = jnp.dot(x_ref[...], w_ref[...], preferred_element_type=F32)
    r = r * dinv_ref[...]
    o_ref[0] = r[:, :DH].astype(BF16)
    o_ref[1] = r[:, DH:].astype(BF16)


def _matmul_split(xp, dinv, w):
    bm = 1024
    out = pl.pallas_call(
        _mm_body,
        grid=(NP // bm,),
        in_specs=[
            pl.BlockSpec((bm, D), lambda i: (i, 0)),
            pl.BlockSpec((bm, 1), lambda i: (i, 0)),
            pl.BlockSpec((D, D), lambda i: (0, 0)),
        ],
        out_specs=pl.BlockSpec((2, bm, DH), lambda i: (0, i, 0)),
        out_shape=jax.ShapeDtypeStruct((2, NP, DH), BF16),
    )(xp, dinv, w)
    return out.reshape(2 * NP, DH)


def _scb_body(xw_hbm, row_hbm, col_hbm, ew_hbm, b1_hbm, w2_hbm, dinv_hbm,
              z_out,
              row_l, col_l, ew_l, tmp_l, zb_l, gbuf, fbuf, b1_l, w2_l,
              h_sh, gsem, ssem):
    c = lax.axis_index("c")
    s = lax.axis_index("s")

    pltpu.sync_copy(b1_hbm.at[c], b1_l)
    pltpu.sync_copy(w2_hbm.at[c], w2_l)
    pltpu.sync_copy(dinv_hbm.at[s], tmp_l)

    def cvt_scale_row(src_r, dst_r, sv):
        for kq in range(DH // 32):
            bb = gbuf[src_r, pl.ds(kq * 32, 32)]
            a, b = plsc.unpack(bb, format=plsc.PackFormat.INTERLEAVED)
            if sv is not None:
                a = a * sv
                b = b * sv
            fbuf[dst_r, pl.ds(kq * 32, 16)] = a
            fbuf[dst_r, pl.ds(kq * 32 + 16, 16)] = b

    for ch in range(NRT // EC):
        base = s * NRT + ch * EC
        pltpu.sync_copy(xw_hbm.at[pl.ds(c * NP + base, EC)],
                        gbuf.at[pl.ds(0, EC)])

        def sl_step(r, _):
            cvt_scale_row(r, r, None)
            return 0
        lax.fori_loop(0, EC, sl_step, 0)
        pltpu.sync_copy(fbuf.at[pl.ds(0, EC)], h_sh.at[pl.ds(base, EC)])
    plsc.subcore_barrier()

    def stage_chunk(g):
        h = (g % 2) * SR
        gbase = s * RPT + g * SR
        pltpu.sync_copy(row_hbm.at[pl.ds(gbase, SR)],
                        row_l.at[pl.ds(h, SR)])
        pltpu.sync_copy(col_hbm.at[pl.ds(gbase, SR)],
                        col_l.at[pl.ds(h, SR)])
        pltpu.sync_copy(ew_hbm.at[pl.ds(gbase, SR)],
                        ew_l.at[pl.ds(h, SR)])

        def adj_step(j, _):
            for k in range(EC // 16):
                rv = row_l[h + j, pl.ds(k * 16, 16)]
                row_l[h + j, pl.ds(k * 16, 16)] = rv + c * NP
            return 0
        lax.fori_loop(0, SR, adj_step, 0)

    def fire_gather(k, p):
        g = k // SR
        j = (g % 2) * SR + (k - g * SR)
        pltpu.async_copy(xw_hbm.at[row_l.at[j]],
                         gbuf.at[pl.ds(p * EC, EC)], gsem.at[p])

    def wait_gather(p):
        pltpu.make_async_copy(xw_hbm.at[pl.ds(0, EC)],
                              gbuf.at[pl.ds(p * EC, EC)], gsem.at[p]).wait()

    def wait_scatter(p):
        pltpu.make_async_copy(xw_hbm.at[pl.ds(0, EC)],
                              fbuf.at[pl.ds(p * EC, EC)], ssem.at[p]).wait()

    stage_chunk(0)
    fire_gather(0, 0)

    def agg_step(k, _):
        p = lax.rem(k, 2)
        q = 1 - p
        g = k // SR
        j = (g % 2) * SR + (k - g * SR)
        last_in_sc = (k - g * SR) == (SR - 1)
        wait_gather(p)

        @pl.when(jnp.logical_and(k + 1 < RPT, jnp.logical_not(last_in_sc)))
        def _():
            fire_gather(k + 1, q)

        @pl.when(k >= 2)
        def _():
            wait_scatter(p)

        def scale_step(jj, _):
            ev = plsc.load_gather(
                ew_l, [jnp.full((16,), j, jnp.int32),
                       jnp.full((16,), jj, jnp.int32)])
            cvt_scale_row(p * EC + jj, p * EC + jj, ev)
            return 0
        lax.fori_loop(0, EC, scale_step, 0)

        pltpu.async_copy(fbuf.at[pl.ds(p * EC, EC)], h_sh.at[col_l.at[j]],
                         ssem.at[p], add=True)

        @pl.when(jnp.logical_and(last_in_sc, k + 1 < RPT))
        def _():
            stage_chunk(g + 1)
            fire_gather(k + 1, q)
        return 0
    lax.fori_loop(0, RPT, agg_step, 0)

    wait_scatter(1)
    wait_scatter(0)
    plsc.subcore_barrier()

    for ch in range(NRT // EC):
        base = s * NRT + ch * EC
        pltpu.sync_copy(h_sh.at[pl.ds(base, EC)], fbuf.at[pl.ds(0, EC)])

        def z_step(r, _):
            dv = plsc.load_gather(
                tmp_l, [jnp.full((16,), ch * EC + r, jnp.int32)])
            acc = jnp.zeros((16,), F32)
            for v in range(DH // 16):
                hv = fbuf[r, pl.ds(v * 16, 16)] * dv + b1_l[pl.ds(v * 16, 16)]
                hv = jnp.maximum(hv, 0.0)
                acc = acc + hv * w2_l[pl.ds(v * 16, 16)]
            zv = jnp.full((16,), jnp.sum(acc), F32)
            plsc.store_scatter(zb_l, [jnp.full((16,), ch * EC + r,
                                               jnp.int32)], zv,
                               mask=lax.iota(jnp.int32, 16) == 0)
            return 0
        lax.fori_loop(0, EC, z_step, 0)

    pltpu.sync_copy(zb_l, z_out.at[c, s])


def _sc_phase1(xw01, row2, col2, ew2, b1p, w2p, dinvo):
    mesh = plsc.VectorSubcoreMesh(core_axis_name="c", subcore_axis_name="s")
    f = pl.kernel(
        _scb_body,
        out_type=jax.ShapeDtypeStruct((2, 16, NRT), F32),
        mesh=mesh,
        compiler_params=_SC_PARAMS,
        scratch_types=[
            pltpu.VMEM((2 * SR, EC), jnp.int32),
            pltpu.VMEM((2 * SR, EC), jnp.int32),
            pltpu.VMEM((2 * SR, EC), F32),
            pltpu.VMEM((NRT,), F32),
            pltpu.VMEM((NRT,), F32),
            pltpu.VMEM((2 * EC, DH), BF16),
            pltpu.VMEM((2 * EC, DH), F32),
            pltpu.VMEM((DH,), F32),
            pltpu.VMEM((DH,), F32),
            pltpu.VMEM_SHARED((NP, DH), F32),
            pltpu.SemaphoreType.DMA((2,)),
            pltpu.SemaphoreType.DMA((2,)),
        ],
    )
    return f(xw01, row2, col2, ew2, b1p, w2p, dinvo)


def _scc_body(z_hbm, row_hbm, col_hbm, ew_hbm, dinv_hbm, b2_hbm,
              out_hbm,
              row_l, col_l, ewn_l, mv_l, z_l, dinv_l, red_l, b2_l,
              o_sh, msem):
    c = lax.axis_index("c")
    s = lax.axis_index("s")

    pltpu.sync_copy(z_hbm.at[0], z_l)
    pltpu.sync_copy(z_hbm.at[1], dinv_l)
    pltpu.sync_copy(b2_hbm, b2_l)

    def zsum_step(i, _):
        z_l[pl.ds(i * 16, 16)] = (z_l[pl.ds(i * 16, 16)]
                                  + dinv_l[pl.ds(i * 16, 16)])
        return 0
    lax.fori_loop(0, NP // 16, zsum_step, 0)

    pltpu.sync_copy(dinv_hbm, dinv_l)

    def oinit_step(i, _):
        dv = dinv_l[pl.ds(s * NRT + i * 16, 16)]
        red_l[pl.ds(i * 16, 16)] = (dv * dv
                                    * z_l[pl.ds(s * NRT + i * 16, 16)])
        return 0
    lax.fori_loop(0, NRT // 16, oinit_step, 0)
    pltpu.sync_copy(red_l, o_sh.at[pl.ds(s * NRT, NRT)])
    plsc.subcore_barrier()

    def msg_chunk(g, _):
        h = (g % 2) * SR
        gbase = s * RPT + g * SR
        pltpu.sync_copy(row_hbm.at[pl.ds(gbase, SR)],
                        row_l.at[pl.ds(h, SR)])
        pltpu.sync_copy(col_hbm.at[pl.ds(gbase, SR)],
                        col_l.at[pl.ds(h, SR)])
        pltpu.sync_copy(ew_hbm.at[pl.ds(gbase, SR)],
                        ewn_l.at[pl.ds(h, SR)])

        def msg_step(j, _):
            for k in range(EC // 16):
                rv = row_l[h + j, pl.ds(k * 16, 16)]
                cv = col_l[h + j, pl.ds(k * 16, 16)]
                wv = ewn_l[h + j, pl.ds(k * 16, 16)]
                dr = plsc.load_gather(dinv_l, [rv])
                dc = plsc.load_gather(dinv_l, [cv])
                zg = plsc.load_gather(z_l, [rv])
                mv_l[h + j, pl.ds(k * 16, 16)] = dr * wv * dc * zg
            return 0
        lax.fori_loop(0, SR, msg_step, 0)

        def agg_fire(j, _):
            pltpu.async_copy(mv_l.at[h + j], o_sh.at[col_l.at[h + j]],
                             msem.at[0], add=True)
            return 0
        lax.fori_loop(0, SR, agg_fire, 0)

        def agg_drain(j, _):
            pltpu.make_async_copy(mv_l.at[h + j], o_sh.at[pl.ds(0, EC)],
                                  msem.at[0]).wait()
            return 0
        lax.fori_loop(0, SR, agg_drain, 0)
        return 0
    lax.fori_loop(0, NSC, msg_chunk, 0)
    plsc.subcore_barrier()

    pltpu.sync_copy(o_sh.at[pl.ds(s * NRT, NRT)], red_l)

    def out_step(i, _):
        o = red_l[pl.ds(i * 16, 16)] + b2_l[...]
        red_l[pl.ds(i * 16, 16)] = 1.0 / (1.0 + jnp.exp(-o))
        return 0
    lax.fori_loop(0, NRT // 16, out_step, 0)

    @pl.when(c == 0)
    def _():
        pltpu.sync_copy(red_l, out_hbm.at[s])


def _sc_phase2(z2, row2, col2, ew2, dinv, b2b):
    mesh = plsc.VectorSubcoreMesh(core_axis_name="c", subcore_axis_name="s")
    f = pl.kernel(
        _scc_body,
        out_type=jax.ShapeDtypeStruct((16, NRT), F32),
        mesh=mesh,
        compiler_params=_SC_PARAMS,
        scratch_types=[
            pltpu.VMEM((2 * SR, EC), jnp.int32),
            pltpu.VMEM((2 * SR, EC), jnp.int32),
            pltpu.VMEM((2 * SR, EC), F32),
            pltpu.VMEM((2 * SR, EC), F32),
            pltpu.VMEM((NP,), F32),
            pltpu.VMEM((NP,), F32),
            pltpu.VMEM((NRT,), F32),
            pltpu.VMEM((16,), F32),
            pltpu.VMEM_SHARED((NP,), F32),
            pltpu.SemaphoreType.DMA((1,)),
        ],
    )
    return f(z2, row2, col2, ew2, dinv, b2b)


def kernel(x, edge_index, edge_attr, W1, b1, W2, b2):
    row2 = edge_index[0].astype(jnp.int32).reshape(ER, EC)
    col2 = edge_index[1].astype(jnp.int32).reshape(ER, EC)
    ew2 = edge_attr.reshape(ER, EC)
    dinvo = _sc_deg(col2, ew2)
    dinv = dinvo.reshape(NP)
    xp = jnp.pad(x, ((0, NP - N), (0, 0)))
    xw01 = _matmul_split(xp, dinv.reshape(NP, 1), W1)
    perm = jnp.asarray(_PERM)
    b1h = b1.reshape(2, DH)[:, perm]
    w2h = W2.reshape(2, DH)[:, perm]
    zparts = _sc_phase1(xw01, row2, col2, ew2, b1h, w2h, dinvo)
    z2 = zparts.reshape(2, NP)
    b2b = jnp.broadcast_to(b2, (16,))
    outp = _sc_phase2(z2, row2, col2, ew2, dinv, b2b)
    return outp.reshape(NP)[:N]


# speedup vs baseline: 1.4134x; 1.4134x over previous
import jax
import jax.numpy as jnp
from jax import lax
from jax.experimental import pallas as pl
from jax.experimental.pallas import tpu as pltpu
from jax.experimental.pallas import tpu_sc as plsc

N = 10000
E = 160000
NP = 10240
D = 256
DH = 128
ER = 2000
EC = 80
SR = 25
NSC = 5
RPT = ER // 16
NRT = NP // 16
F32 = jnp.float32

_SC_PARAMS = pltpu.CompilerParams(use_tc_tiling_on_sc=False,
                                  needs_layout_passes=False)


def _newton_rsqrt(d):
    i = plsc.bitcast(d, jnp.int32)
    i = jnp.int32(0x5F3759DF) - lax.shift_right_logical(i, 1)
    y = plsc.bitcast(i, F32)
    for _ in range(4):
        y = y * (1.5 - 0.5 * d * y * y)
    return y


def _sca_body(col_hbm, ew_hbm, dinv_out,
              col_l, ew_l, tmp_l, dd_sh, sem):
    c = lax.axis_index("c")
    s = lax.axis_index("s")

    def ones_step(i, _):
        tmp_l[pl.ds(i * 16, 16)] = jnp.full((16,), 1.0, F32)
        return 0
    lax.fori_loop(0, NRT // 16, ones_step, 0)
    pltpu.sync_copy(tmp_l, dd_sh.at[pl.ds(s * NRT, NRT)])
    plsc.subcore_barrier()

    def deg_chunk(g, _):
        gbase = s * RPT + g * SR
        pltpu.sync_copy(col_hbm.at[pl.ds(gbase, SR)], col_l)
        pltpu.sync_copy(ew_hbm.at[pl.ds(gbase, SR)], ew_l)

        def deg_fire(j, _):
            pltpu.async_copy(ew_l.at[j], dd_sh.at[col_l.at[j]],
                             sem, add=True)
            return 0
        lax.fori_loop(0, SR, deg_fire, 0)

        def deg_drain(j, _):
            pltpu.make_async_copy(ew_l.at[j], dd_sh.at[pl.ds(0, EC)],
                                  sem).wait()
            return 0
        lax.fori_loop(0, SR, deg_drain, 0)
        return 0
    lax.fori_loop(0, NSC, deg_chunk, 0)
    plsc.subcore_barrier()

    pltpu.sync_copy(dd_sh.at[pl.ds(s * NRT, NRT)], tmp_l)

    def dinv_step(i, _):
        d = tmp_l[pl.ds(i * 16, 16)]
        tmp_l[pl.ds(i * 16, 16)] = _newton_rsqrt(d)
        return 0
    lax.fori_loop(0, NRT // 16, dinv_step, 0)

    @pl.when(c == 0)
    def _():
        pltpu.sync_copy(tmp_l, dinv_out.at[s])


def _sc_deg(col2, ew2):
    mesh = plsc.VectorSubcoreMesh(core_axis_name="c", subcore_axis_name="s")
    f = pl.kernel(
        _sca_body,
        out_type=jax.ShapeDtypeStruct((16, NRT), F32),
        mesh=mesh,
        compiler_params=_SC_PARAMS,
        scratch_types=[
            pltpu.VMEM((SR, EC), jnp.int32),
            pltpu.VMEM((SR, EC), F32),
            pltpu.VMEM((NRT,), F32),
            pltpu.VMEM_SHARED((NP,), F32),
            pltpu.SemaphoreType.DMA,
        ],
    )
    return f(col2, ew2)


def _mm_body(x_ref, dinv_ref, w_ref, o_ref):
    r = jnp.dot(x_ref[...], w_ref[...], preferred_element_type=F32)
    r = r * dinv_ref[...]
    o_ref[0] = r[:, :DH].astype(F32)
    o_ref[1] = r[:, DH:].astype(F32)


def _matmul_split(xp, dinv, w):
    bm = 1024
    out = pl.pallas_call(
        _mm_body,
        grid=(NP // bm,),
        in_specs=[
            pl.BlockSpec((bm, D), lambda i: (i, 0)),
            pl.BlockSpec((bm, 1), lambda i: (i, 0)),
            pl.BlockSpec((D, D), lambda i: (0, 0)),
        ],
        out_specs=pl.BlockSpec((2, bm, DH), lambda i: (0, i, 0)),
        out_shape=jax.ShapeDtypeStruct((2, NP, DH), F32),
    )(xp, dinv, w)
    return out.reshape(2 * NP, DH)


def _scb_body(xw_hbm, row_hbm, col_hbm, ew_hbm, b1_hbm, w2_hbm, dinv_hbm,
              z_out,
              row_l, col_l, ew_l, tmp_l, zb_l, gbuf, b1_l, w2_l,
              h_sh, gsem, ssem):
    c = lax.axis_index("c")
    s = lax.axis_index("s")

    pltpu.sync_copy(b1_hbm.at[c], b1_l)
    pltpu.sync_copy(w2_hbm.at[c], w2_l)
    pltpu.sync_copy(dinv_hbm.at[s], tmp_l)

    def cvt_scale_row(src_r, dst_r, sv):
        for kq in range(DH // 16):
            bb = gbuf[src_r, pl.ds(kq * 16, 16)]
            gbuf[dst_r, pl.ds(kq * 16, 16)] = bb * sv

    for ch in range(NRT // EC):
        base = s * NRT + ch * EC
        pltpu.sync_copy(xw_hbm.at[pl.ds(c * NP + base, EC)],
                        h_sh.at[pl.ds(base, EC)])
    plsc.subcore_barrier()

    def stage_chunk(g):
        h = (g % 2) * SR
        gbase = s * RPT + g * SR
        pltpu.sync_copy(row_hbm.at[pl.ds(gbase, SR)],
                        row_l.at[pl.ds(h, SR)])
        pltpu.sync_copy(col_hbm.at[pl.ds(gbase, SR)],
                        col_l.at[pl.ds(h, SR)])
        pltpu.sync_copy(ew_hbm.at[pl.ds(gbase, SR)],
                        ew_l.at[pl.ds(h, SR)])

        def adj_step(j, _):
            for k in range(EC // 16):
                rv = row_l[h + j, pl.ds(k * 16, 16)]
                row_l[h + j, pl.ds(k * 16, 16)] = rv + c * NP
            return 0
        lax.fori_loop(0, SR, adj_step, 0)

    def fire_gather(k, p):
        g = k // SR
        j = (g % 2) * SR + (k - g * SR)
        pltpu.async_copy(xw_hbm.at[row_l.at[j]],
                         gbuf.at[pl.ds(p * EC, EC)], gsem.at[p])

    def wait_gather(p):
        pltpu.make_async_copy(xw_hbm.at[pl.ds(0, EC)],
                              gbuf.at[pl.ds(p * EC, EC)], gsem.at[p]).wait()

    def wait_scatter(p):
        pltpu.make_async_copy(xw_hbm.at[pl.ds(0, EC)],
                              gbuf.at[pl.ds(p * EC, EC)], ssem.at[p]).wait()

    stage_chunk(0)
    fire_gather(0, 0)

    def agg_step(k, _):
        p = lax.rem(k, 2)
        q = 1 - p
        g = k // SR
        j = (g % 2) * SR + (k - g * SR)
        last_in_sc = (k - g * SR) == (SR - 1)
        wait_gather(p)

        @pl.when(jnp.logical_and(k + 1 < RPT, jnp.logical_not(last_in_sc)))
        def _():
            @pl.when(k >= 1)
            def _():
                wait_scatter(q)
            fire_gather(k + 1, q)

        def scale_step(jj, _):
            ev = plsc.load_gather(
                ew_l, [jnp.full((16,), j, jnp.int32),
                       jnp.full((16,), jj, jnp.int32)])
            cvt_scale_row(p * EC + jj, p * EC + jj, ev)
            return 0
        lax.fori_loop(0, EC, scale_step, 0)

        pltpu.async_copy(gbuf.at[pl.ds(p * EC, EC)], h_sh.at[col_l.at[j]],
                         ssem.at[p], add=True)

        @pl.when(jnp.logical_and(last_in_sc, k + 1 < RPT))
        def _():
            stage_chunk(g + 1)

            @pl.when(k >= 1)
            def _():
                wait_scatter(q)
            fire_gather(k + 1, q)
        return 0
    lax.fori_loop(0, RPT, agg_step, 0)

    wait_scatter(1)
    wait_scatter(0)
    plsc.subcore_barrier()

    for ch in range(NRT // EC):
        base = s * NRT + ch * EC
        pltpu.sync_copy(h_sh.at[pl.ds(base, EC)], gbuf.at[pl.ds(0, EC)])

        def z_step(r, _):
            dv = plsc.load_gather(
                tmp_l, [jnp.full((16,), ch * EC + r, jnp.int32)])
            acc = jnp.zeros((16,), F32)
            for v in range(DH // 16):
                hv = gbuf[r, pl.ds(v * 16, 16)] * dv + b1_l[pl.ds(v * 16, 16)]
                hv = jnp.maximum(hv, 0.0)
                acc = acc + hv * w2_l[pl.ds(v * 16, 16)]
            zv = jnp.full((16,), jnp.sum(acc), F32)
            plsc.store_scatter(zb_l, [jnp.full((16,), ch * EC + r,
                                               jnp.int32)], zv,
                               mask=lax.iota(jnp.int32, 16) == 0)
            return 0
        lax.fori_loop(0, EC, z_step, 0)

    pltpu.sync_copy(zb_l, z_out.at[c, s])


def _sc_phase1(xw01, row2, col2, ew2, b1p, w2p, dinvo):
    mesh = plsc.VectorSubcoreMesh(core_axis_name="c", subcore_axis_name="s")
    f = pl.kernel(
        _scb_body,
        out_type=jax.ShapeDtypeStruct((2, 16, NRT), F32),
        mesh=mesh,
        compiler_params=_SC_PARAMS,
        scratch_types=[
            pltpu.VMEM((2 * SR, EC), jnp.int32),
            pltpu.VMEM((2 * SR, EC), jnp.int32),
            pltpu.VMEM((2 * SR, EC), F32),
            pltpu.VMEM((NRT,), F32),
            pltpu.VMEM((NRT,), F32),
            pltpu.VMEM((2 * EC, DH), F32),
            pltpu.VMEM((DH,), F32),
            pltpu.VMEM((DH,), F32),
            pltpu.VMEM_SHARED((NP, DH), F32),
            pltpu.SemaphoreType.DMA((2,)),
            pltpu.SemaphoreType.DMA((2,)),
        ],
    )
    return f(xw01, row2, col2, ew2, b1p, w2p, dinvo)


def _scc_body(z_hbm, row_hbm, col_hbm, ew_hbm, dinv_hbm, b2_hbm,
              out_hbm,
              row_l, col_l, ewn_l, mv_l, z_l, dinv_l, red_l, b2_l,
              o_sh, msem):
    c = lax.axis_index("c")
    s = lax.axis_index("s")

    pltpu.sync_copy(z_hbm.at[0], z_l)
    pltpu.sync_copy(z_hbm.at[1], dinv_l)
    pltpu.sync_copy(b2_hbm, b2_l)

    def zsum_step(i, _):
        z_l[pl.ds(i * 16, 16)] = (z_l[pl.ds(i * 16, 16)]
                                  + dinv_l[pl.ds(i * 16, 16)])
        return 0
    lax.fori_loop(0, NP // 16, zsum_step, 0)

    pltpu.sync_copy(dinv_hbm, dinv_l)

    def oinit_step(i, _):
        dv = dinv_l[pl.ds(s * NRT + i * 16, 16)]
        red_l[pl.ds(i * 16, 16)] = (dv * dv
                                    * z_l[pl.ds(s * NRT + i * 16, 16)])
        return 0
    lax.fori_loop(0, NRT // 16, oinit_step, 0)
    pltpu.sync_copy(red_l, o_sh.at[pl.ds(s * NRT, NRT)])
    plsc.subcore_barrier()

    def msg_chunk(g, _):
        h = (g % 2) * SR
        gbase = s * RPT + g * SR
        pltpu.sync_copy(row_hbm.at[pl.ds(gbase, SR)],
                        row_l.at[pl.ds(h, SR)])
        pltpu.sync_copy(col_hbm.at[pl.ds(gbase, SR)],
                        col_l.at[pl.ds(h, SR)])
        pltpu.sync_copy(ew_hbm.at[pl.ds(gbase, SR)],
                        ewn_l.at[pl.ds(h, SR)])

        def msg_step(j, _):
            for k in range(EC // 16):
                rv = row_l[h + j, pl.ds(k * 16, 16)]
                cv = col_l[h + j, pl.ds(k * 16, 16)]
                wv = ewn_l[h + j, pl.ds(k * 16, 16)]
                dr = plsc.load_gather(dinv_l, [rv])
                dc = plsc.load_gather(dinv_l, [cv])
                zg = plsc.load_gather(z_l, [rv])
                mv_l[h + j, pl.ds(k * 16, 16)] = dr * wv * dc * zg
            return 0
        lax.fori_loop(0, SR, msg_step, 0)

        def agg_fire(j, _):
            pltpu.async_copy(mv_l.at[h + j], o_sh.at[col_l.at[h + j]],
                             msem.at[0], add=True)
            return 0
        lax.fori_loop(0, SR, agg_fire, 0)

        def agg_drain(j, _):
            pltpu.make_async_copy(mv_l.at[h + j], o_sh.at[pl.ds(0, EC)],
                                  msem.at[0]).wait()
            return 0
        lax.fori_loop(0, SR, agg_drain, 0)
        return 0
    lax.fori_loop(0, NSC, msg_chunk, 0)
    plsc.subcore_barrier()

    pltpu.sync_copy(o_sh.at[pl.ds(s * NRT, NRT)], red_l)

    def out_step(i, _):
        o = red_l[pl.ds(i * 16, 16)] + b2_l[...]
        red_l[pl.ds(i * 16, 16)] = 1.0 / (1.0 + jnp.exp(-o))
        return 0
    lax.fori_loop(0, NRT // 16, out_step, 0)

    @pl.when(c == 0)
    def _():
        pltpu.sync_copy(red_l, out_hbm.at[s])


def _sc_phase2(z2, row2, col2, ew2, dinv, b2b):
    mesh = plsc.VectorSubcoreMesh(core_axis_name="c", subcore_axis_name="s")
    f = pl.kernel(
        _scc_body,
        out_type=jax.ShapeDtypeStruct((16, NRT), F32),
        mesh=mesh,
        compiler_params=_SC_PARAMS,
        scratch_types=[
            pltpu.VMEM((2 * SR, EC), jnp.int32),
            pltpu.VMEM((2 * SR, EC), jnp.int32),
            pltpu.VMEM((2 * SR, EC), F32),
            pltpu.VMEM((2 * SR, EC), F32),
            pltpu.VMEM((NP,), F32),
            pltpu.VMEM((NP,), F32),
            pltpu.VMEM((NRT,), F32),
            pltpu.VMEM((16,), F32),
            pltpu.VMEM_SHARED((NP,), F32),
            pltpu.SemaphoreType.DMA((1,)),
        ],
    )
    return f(z2, row2, col2, ew2, dinv, b2b)


def kernel(x, edge_index, edge_attr, W1, b1, W2, b2):
    row2 = edge_index[0].astype(jnp.int32).reshape(ER, EC)
    col2 = edge_index[1].astype(jnp.int32).reshape(ER, EC)
    ew2 = edge_attr.reshape(ER, EC)
    dinvo = _sc_deg(col2, ew2)
    dinv = dinvo.reshape(NP)
    xp = jnp.pad(x, ((0, NP - N), (0, 0)))
    xw01 = _matmul_split(xp, dinv.reshape(NP, 1), W1)
    b1h = b1.reshape(2, DH)
    w2h = W2.reshape(2, DH)
    zparts = _sc_phase1(xw01, row2, col2, ew2, b1h, w2h, dinvo)
    z2 = zparts.reshape(2, NP)
    b2b = jnp.broadcast_to(b2, (16,))
    outp = _sc_phase2(z2, row2, col2, ew2, dinv, b2b)
    return outp.reshape(NP)[:N]
